# baseline (speedup 1.0000x reference)
import jax
import jax.numpy as jnp
from jax import lax
from jax.experimental import pallas as pl
from jax.experimental.pallas import tpu as pltpu
from jax.experimental.pallas import tpu_sc as plsc

_N = 10000
_E = 320000
_D = 128

_NC = 2
_NS = 16
_NW = _NC * _NS
_EPT = _E // _NW
_CHUNK = 80
_ITERS = _EPT // _CHUNK
_NP = 10240
_RPT = _NP // _NS
_LANES = 16


def _sc_segsum(with_counts):
    mesh = plsc.VectorSubcoreMesh(
        core_axis_name="c", subcore_axis_name="s",
        num_cores=_NC, num_subcores=_NS)

    out_type = [jax.ShapeDtypeStruct((_NC, _NP, _D), jnp.float32)]
    scratch = [
        pltpu.VMEM((_EPT,), jnp.int32),
        pltpu.VMEM((_CHUNK,), jnp.int32),
        pltpu.VMEM((_CHUNK,), jnp.int32),
        pltpu.VMEM((_CHUNK,), jnp.int32),
        pltpu.VMEM((_CHUNK, _D), jnp.float32),
        pltpu.VMEM((_CHUNK, _D), jnp.float32),
        pltpu.VMEM((_CHUNK, _D), jnp.float32),
        pltpu.VMEM_SHARED((_NP, _D), jnp.float32),
        pltpu.SemaphoreType.DMA,
        pltpu.SemaphoreType.DMA,
        pltpu.SemaphoreType.DMA,
        pltpu.SemaphoreType.DMA,
        pltpu.SemaphoreType.DMA,
        pltpu.SemaphoreType.DMA,
    ]
    if with_counts:
        out_type.append(jax.ShapeDtypeStruct((_NC, _NP), jnp.float32))
        scratch += [
            pltpu.VMEM((_CHUNK,), jnp.float32),
            pltpu.VMEM((_RPT,), jnp.float32),
            pltpu.VMEM_SHARED((_NP,), jnp.float32),
            pltpu.SemaphoreType.DMA,
            pltpu.SemaphoreType.DMA,
            pltpu.SemaphoreType.DMA,
        ]

    def body(x_hbm, ei_hbm, *refs):
        if with_counts:
            (agg_out, cnt_out, isrc, ib0, ib1, ib2, rows0, rows1, rows2,
             acc, g0, g1, g2, s0, s1, s2, ones_v, zv, cacc,
             c0, c1, c2) = refs
            csems = (c0, c1, c2)
        else:
            (agg_out, isrc, ib0, ib1, ib2, rows0, rows1, rows2,
             acc, g0, g1, g2, s0, s1, s2) = refs
        ibs = (ib0, ib1, ib2)
        rows = (rows0, rows1, rows2)
        gsems = (g0, g1, g2)
        ssems = (s0, s1, s2)

        c = lax.axis_index("c")
        s = lax.axis_index("s")
        w = c * _NS + s
        ebase = pl.multiple_of(w * _EPT, 8)
        row0 = s * _RPT
        zero16 = jnp.zeros((_LANES,), jnp.float32)

        def issue_stage(chunk, u):
            off = pl.multiple_of(_E + w * _EPT + chunk * _CHUNK, 8)
            pltpu.async_copy(ei_hbm.at[pl.ds(off, _CHUNK)], ibs[u], gsems[u])

        def issue_gather(chunk, u):
            off = pl.multiple_of(chunk * _CHUNK, 8)
            pltpu.async_copy(x_hbm.at[isrc.at[pl.ds(off, _CHUNK)]],
                             rows[u], gsems[u])

        def wait_pair(u):
            pltpu.make_async_copy(
                ei_hbm.at[pl.ds(0, _CHUNK)], ibs[u], gsems[u]).wait()
            pltpu.make_async_copy(
                x_hbm.at[isrc.at[pl.ds(0, _CHUNK)]], rows[u],
                gsems[u]).wait()

        def issue_scatter(u):
            if with_counts:
                pltpu.async_copy(ones_v, cacc.at[ibs[u]], csems[u],
                                 add=True)
            pltpu.async_copy(rows[u], acc.at[ibs[u]], ssems[u], add=True)

        def wait_scatter(u):
            pltpu.make_async_copy(rows[u], acc.at[ibs[u]], ssems[u]).wait()
            if with_counts:
                pltpu.make_async_copy(ones_v, cacc.at[ibs[u]],
                                      csems[u]).wait()

        pltpu.async_copy(ei_hbm.at[pl.ds(ebase, _EPT)], isrc, g0)

        def zrow(j, _):
            for k in range(_D // _LANES):
                rows2[j, pl.ds(k * _LANES, _LANES)] = zero16
            return 0
        lax.fori_loop(0, _CHUNK, zrow, 0)

        if with_counts:
            def fill(j, _):
                ones_v[pl.ds(j * _LANES, _LANES)] = zero16 + 1.0
                return 0
            lax.fori_loop(0, _CHUNK // _LANES, fill, 0)

            def zc(j, _):
                zv[pl.ds(j * _LANES, _LANES)] = zero16
                return 0
            lax.fori_loop(0, _RPT // _LANES, zc, 0)

        pltpu.make_async_copy(ei_hbm.at[pl.ds(0, _EPT)], isrc, g0).wait()
        issue_stage(0, 0)
        issue_gather(0, 0)
        issue_stage(1, 1)
        issue_gather(1, 1)

        for off in range(0, _RPT, _CHUNK):
            pltpu.async_copy(rows2, acc.at[pl.ds(row0 + off, _CHUNK)], s0)
        if with_counts:
            pltpu.sync_copy(zv, cacc.at[pl.ds(row0, _RPT)])
        for _ in range(_RPT // _CHUNK):
            pltpu.make_async_copy(
                rows2, acc.at[pl.ds(row0, _CHUNK)], s0).wait()
        plsc.subcore_barrier()

        def step(k, _):
            for u in range(3):
                chunk = 3 * k + u

                @pl.when(chunk < _ITERS)
                def _():
                    wait_pair(u)
                    issue_scatter(u)

                nu = (u + 2) % 3

                @pl.when(chunk + 2 < _ITERS)
                def _():
                    if u == 0:
                        @pl.when(k >= 1)
                        def _():
                            wait_scatter(nu)
                    else:
                        wait_scatter(nu)
                    issue_stage(chunk + 2, nu)
                    issue_gather(chunk + 2, nu)
            return 0
        lax.fori_loop(0, (_ITERS + 4) // 3, step, 0)

        for u in range(3):
            wait_scatter(u)

        plsc.subcore_barrier()

        pltpu.sync_copy(acc.at[pl.ds(row0, _RPT)],
                        agg_out.at[c, pl.ds(row0, _RPT)])
        if with_counts:
            pltpu.sync_copy(cacc.at[pl.ds(row0, _RPT)],
                            cnt_out.at[c, pl.ds(row0, _RPT)])

    return pl.kernel(body, out_type=out_type, mesh=mesh,
                     scratch_types=scratch,
                     name="sc_segsum_cnt" if with_counts else "sc_segsum")


_RB = 2000
_GRID = _N // _RB


def _tcr_body(x, wr, b, out_ref):
    out_ref[...] = (jnp.dot(x[...], wr[...],
                            preferred_element_type=jnp.float32) + b[...])


def _tc1_body(agg, cnt, xr, wl, h_out, inv_out):
    s = agg[0] + agg[1]
    ctot = cnt[0] + cnt[1]
    inv = 1.0 / jnp.maximum(ctot, 1.0)
    mean = s * inv
    out = jnp.dot(mean, wl[...], preferred_element_type=jnp.float32) + xr[...]
    h_out[...] = jnp.maximum(out, 0.0)
    inv_out[...] = inv


def _tc2_body(agg, inv, hr, wl, out_ref):
    s = agg[0] + agg[1]
    mean = s * inv[...]
    out = jnp.dot(mean, wl[...], preferred_element_type=jnp.float32) + hr[...]
    nrm = jnp.sqrt(jnp.sum(out * out, axis=1, keepdims=True))
    out = out / jnp.maximum(nrm, 1e-12)
    out_ref[...] = jnp.maximum(out, 0.0)


_row_spec = pl.BlockSpec((_RB, _D), lambda i: (i, 0))
_w_spec = pl.BlockSpec((_D, _D), lambda i: (0, 0))

_tcr = pl.pallas_call(
    _tcr_body,
    grid=(_GRID,),
    in_specs=[_row_spec, _w_spec, pl.BlockSpec((1, _D), lambda i: (0, 0))],
    out_specs=_row_spec,
    out_shape=jax.ShapeDtypeStruct((_N, _D), jnp.float32),
)

_tc1 = pl.pallas_call(
    _tc1_body,
    grid=(_GRID,),
    in_specs=[
        pl.BlockSpec((_NC, _RB, _D), lambda i: (0, i, 0)),
        pl.BlockSpec((_NC, _RB, 1), lambda i: (0, i, 0)),
        _row_spec,
        _w_spec,
    ],
    out_specs=[_row_spec, pl.BlockSpec((_RB, 1), lambda i: (i, 0))],
    out_shape=[
        jax.ShapeDtypeStruct((_N, _D), jnp.float32),
        jax.ShapeDtypeStruct((_N, 1), jnp.float32),
    ],
)

_tc2 = pl.pallas_call(
    _tc2_body,
    grid=(_GRID,),
    in_specs=[
        pl.BlockSpec((_NC, _RB, _D), lambda i: (0, i, 0)),
        pl.BlockSpec((_RB, 1), lambda i: (i, 0)),
        _row_spec,
        _w_spec,
    ],
    out_specs=_row_spec,
    out_shape=jax.ShapeDtypeStruct((_N, _D), jnp.float32),
)

_sc_layer1 = _sc_segsum(with_counts=True)
_sc_layer2 = _sc_segsum(with_counts=False)


@jax.jit
def kernel(matrix_nodes_features, edge_index, W1l, b1, W1r, W2l, b2, W2r):
    x = matrix_nodes_features
    ei = edge_index.astype(jnp.int32).reshape(-1)

    agg1, cnt = _sc_layer1(x, ei)
    xr = _tcr(x, W1r, b1.reshape(1, _D))
    h, inv = _tc1(agg1, cnt.reshape(_NC, _NP, 1), xr, W1l)
    agg2, = _sc_layer2(h, ei)
    hr = _tcr(h, W2r, b2.reshape(1, _D))
    return _tc2(agg2, inv, hr, W2l)


# speedup vs baseline: 1.0063x; 1.0063x over previous
import jax
import jax.numpy as jnp
from jax import lax
from jax.experimental import pallas as pl
from jax.experimental.pallas import tpu as pltpu
from jax.experimental.pallas import tpu_sc as plsc

_N = 10000
_E = 320000
_D = 128

_NC = 2
_NS = 16
_NW = _NC * _NS
_EPT = _E // _NW
_CHUNK = 80
_ITERS = _EPT // _CHUNK
_NP = 10240
_RPT = _NP // _NS
_LANES = 16


def _sc_segsum(with_counts):
    mesh = plsc.VectorSubcoreMesh(
        core_axis_name="c", subcore_axis_name="s",
        num_cores=_NC, num_subcores=_NS)

    out_type = [jax.ShapeDtypeStruct((_NC, _NP, _D), jnp.float32)]
    scratch = [
        pltpu.VMEM((_EPT,), jnp.int32),
        pltpu.VMEM((_CHUNK,), jnp.int32),
        pltpu.VMEM((_CHUNK,), jnp.int32),
        pltpu.VMEM((_CHUNK,), jnp.int32),
        pltpu.VMEM((_CHUNK, _D), jnp.float32),
        pltpu.VMEM((_CHUNK, _D), jnp.float32),
        pltpu.VMEM((_CHUNK, _D), jnp.float32),
        pltpu.VMEM_SHARED((_NP, _D), jnp.float32),
        pltpu.SemaphoreType.DMA,
        pltpu.SemaphoreType.DMA,
        pltpu.SemaphoreType.DMA,
        pltpu.SemaphoreType.DMA,
        pltpu.SemaphoreType.DMA,
        pltpu.SemaphoreType.DMA,
    ]
    if with_counts:
        out_type.append(jax.ShapeDtypeStruct((_NC, _NP), jnp.float32))
        scratch += [
            pltpu.VMEM((_CHUNK,), jnp.float32),
            pltpu.VMEM((_RPT,), jnp.float32),
            pltpu.VMEM_SHARED((_NP,), jnp.float32),
            pltpu.SemaphoreType.DMA,
            pltpu.SemaphoreType.DMA,
            pltpu.SemaphoreType.DMA,
        ]

    def body(x_hbm, ei_hbm, *refs):
        if with_counts:
            (agg_out, cnt_out, isrc, ib0, ib1, ib2, rows0, rows1, rows2,
             acc, g0, g1, g2, s0, s1, s2, ones_v, zv, cacc,
             c0, c1, c2) = refs
            csems = (c0, c1, c2)
        else:
            (agg_out, isrc, ib0, ib1, ib2, rows0, rows1, rows2,
             acc, g0, g1, g2, s0, s1, s2) = refs
        ibs = (ib0, ib1, ib2)
        rows = (rows0, rows1, rows2)
        gsems = (g0, g1, g2)
        ssems = (s0, s1, s2)

        c = lax.axis_index("c")
        s = lax.axis_index("s")
        w = c * _NS + s
        ebase = pl.multiple_of(w * _EPT, 8)
        row0 = s * _RPT
        zero16 = jnp.zeros((_LANES,), jnp.float32)

        def issue_stage(chunk, u):
            off = pl.multiple_of(_E + w * _EPT + chunk * _CHUNK, 8)
            pltpu.async_copy(ei_hbm.at[pl.ds(off, _CHUNK)], ibs[u], gsems[u])

        def issue_gather(chunk, u):
            off = pl.multiple_of(chunk * _CHUNK, 8)
            pltpu.async_copy(x_hbm.at[isrc.at[pl.ds(off, _CHUNK)]],
                             rows[u], gsems[u])

        def wait_pair(u):
            pltpu.make_async_copy(
                ei_hbm.at[pl.ds(0, _CHUNK)], ibs[u], gsems[u]).wait()
            pltpu.make_async_copy(
                x_hbm.at[isrc.at[pl.ds(0, _CHUNK)]], rows[u],
                gsems[u]).wait()

        def issue_scatter(u):
            if with_counts:
                pltpu.async_copy(ones_v, cacc.at[ibs[u]], csems[u],
                                 add=True)
            pltpu.async_copy(rows[u], acc.at[ibs[u]], ssems[u], add=True)

        def wait_scatter(u):
            pltpu.make_async_copy(rows[u], acc.at[ibs[u]], ssems[u]).wait()
            if with_counts:
                pltpu.make_async_copy(ones_v, cacc.at[ibs[u]],
                                      csems[u]).wait()

        pltpu.async_copy(ei_hbm.at[pl.ds(ebase, _EPT)], isrc, g0)

        def zrow(j, _):
            for k in range(_D // _LANES):
                rows2[j, pl.ds(k * _LANES, _LANES)] = zero16
            return 0
        lax.fori_loop(0, _CHUNK, zrow, 0)

        if with_counts:
            def fill(j, _):
                ones_v[pl.ds(j * _LANES, _LANES)] = zero16 + 1.0
                return 0
            lax.fori_loop(0, _CHUNK // _LANES, fill, 0)

            def zc(j, _):
                zv[pl.ds(j * _LANES, _LANES)] = zero16
                return 0
            lax.fori_loop(0, _RPT // _LANES, zc, 0)

        pltpu.make_async_copy(ei_hbm.at[pl.ds(0, _EPT)], isrc, g0).wait()
        issue_stage(0, 0)
        issue_gather(0, 0)
        issue_stage(1, 1)
        issue_gather(1, 1)

        for off in range(0, _RPT, _CHUNK):
            pltpu.async_copy(rows2, acc.at[pl.ds(row0 + off, _CHUNK)], s0)
        if with_counts:
            pltpu.sync_copy(zv, cacc.at[pl.ds(row0, _RPT)])
        for _ in range(_RPT // _CHUNK):
            pltpu.make_async_copy(
                rows2, acc.at[pl.ds(row0, _CHUNK)], s0).wait()
        plsc.subcore_barrier()

        def step(k, _):
            for u in range(3):
                chunk = 3 * k + u

                @pl.when(chunk < _ITERS)
                def _():
                    wait_pair(u)
                    issue_scatter(u)

                nu = (u + 2) % 3

                @pl.when(chunk + 2 < _ITERS)
                def _():
                    if u == 0:
                        @pl.when(k >= 1)
                        def _():
                            wait_scatter(nu)
                    else:
                        wait_scatter(nu)
                    issue_stage(chunk + 2, nu)
                    issue_gather(chunk + 2, nu)
            return 0
        lax.fori_loop(0, (_ITERS + 4) // 3, step, 0)

        for u in range(3):
            wait_scatter(u)

        plsc.subcore_barrier()

        pltpu.sync_copy(acc.at[pl.ds(row0, _RPT)],
                        agg_out.at[c, pl.ds(row0, _RPT)])
        if with_counts:
            pltpu.sync_copy(cacc.at[pl.ds(row0, _RPT)],
                            cnt_out.at[c, pl.ds(row0, _RPT)])

    return pl.kernel(body, out_type=out_type, mesh=mesh,
                     scratch_types=scratch,
                     name="sc_segsum_cnt" if with_counts else "sc_segsum")


_RB = 2000
_GRID = _N // _RB


def _tc1_body(agg, cnt, x, wl, b, wr, h_out, inv_out):
    s = agg[0] + agg[1]
    ctot = cnt[0] + cnt[1]
    inv = 1.0 / jnp.maximum(ctot, 1.0)
    mean = s * inv
    out = (jnp.dot(mean, wl[...], preferred_element_type=jnp.float32)
           + jnp.dot(x[...], wr[...], preferred_element_type=jnp.float32)
           + b[...])
    h_out[...] = jnp.maximum(out, 0.0)
    inv_out[...] = inv


def _tc2_body(agg, inv, h, wl, b, wr, out_ref):
    s = agg[0] + agg[1]
    mean = s * inv[...]
    out = (jnp.dot(mean, wl[...], preferred_element_type=jnp.float32)
           + jnp.dot(h[...], wr[...], preferred_element_type=jnp.float32)
           + b[...])
    nrm = jnp.sqrt(jnp.sum(out * out, axis=1, keepdims=True))
    out = out / jnp.maximum(nrm, 1e-12)
    out_ref[...] = jnp.maximum(out, 0.0)


_row_spec = pl.BlockSpec((_RB, _D), lambda i: (i, 0))
_w_spec = pl.BlockSpec((_D, _D), lambda i: (0, 0))
_b_spec = pl.BlockSpec((1, _D), lambda i: (0, 0))

_tc1 = pl.pallas_call(
    _tc1_body,
    grid=(_GRID,),
    in_specs=[
        pl.BlockSpec((_NC, _RB, _D), lambda i: (0, i, 0)),
        pl.BlockSpec((_NC, _RB, 1), lambda i: (0, i, 0)),
        _row_spec,
        _w_spec,
        _b_spec,
        _w_spec,
    ],
    out_specs=[_row_spec, pl.BlockSpec((_RB, 1), lambda i: (i, 0))],
    out_shape=[
        jax.ShapeDtypeStruct((_N, _D), jnp.float32),
        jax.ShapeDtypeStruct((_N, 1), jnp.float32),
    ],
)

_tc2 = pl.pallas_call(
    _tc2_body,
    grid=(_GRID,),
    in_specs=[
        pl.BlockSpec((_NC, _RB, _D), lambda i: (0, i, 0)),
        pl.BlockSpec((_RB, 1), lambda i: (i, 0)),
        _row_spec,
        _w_spec,
        _b_spec,
        _w_spec,
    ],
    out_specs=_row_spec,
    out_shape=jax.ShapeDtypeStruct((_N, _D), jnp.float32),
)

_sc_layer1 = _sc_segsum(with_counts=True)
_sc_layer2 = _sc_segsum(with_counts=False)


@jax.jit
def kernel(matrix_nodes_features, edge_index, W1l, b1, W1r, W2l, b2, W2r):
    x = matrix_nodes_features
    ei = edge_index.astype(jnp.int32).reshape(-1)

    agg1, cnt = _sc_layer1(x, ei)
    h, inv = _tc1(agg1, cnt.reshape(_NC, _NP, 1), x,
                  W1l, b1.reshape(1, _D), W1r)
    agg2, = _sc_layer2(h, ei)
    return _tc2(agg2, inv, h, W2l, b2.reshape(1, _D), W2r)
